# Initial kernel scaffold; baseline (speedup 1.0000x reference)
import jax
import jax.numpy as jnp
from jax.experimental import pallas as pl


def kernel(indices, table):
    raise NotImplementedError("write your pallas kernel here")


# speedup vs baseline: 1.0415x; 1.0415x over previous
import jax
import jax.numpy as jnp
from jax.experimental import pallas as pl
from jax.experimental.pallas import tpu as pltpu
from jax.experimental.pallas import tpu_sc as plsc

_WINDOW = 128


def kernel(indices, table):
    B, H = indices.shape
    V, D = table.shape
    n = B * H
    flat_idx = indices.reshape(1, n).astype(jnp.int32)
    mesh = plsc.VectorSubcoreMesh(core_axis_name="core", subcore_axis_name="subcore")

    @pl.kernel(
        out_type=jax.ShapeDtypeStruct((n, D), table.dtype),
        mesh=mesh,
        compiler_params=pltpu.CompilerParams(use_tc_tiling_on_sc=False),
    )
    def gather_kernel(x_hbm, i_hbm, o_hbm):
        def body(i_vmem, o_vmem):
            pltpu.sync_copy(x_hbm.at[i_vmem.at[0]], o_vmem)

        pltpu.emit_pipeline(
            body,
            grid=(n // _WINDOW,),
            in_specs=[pl.BlockSpec((1, _WINDOW), index_map=lambda i: (0, i))],
            out_specs=[pl.BlockSpec((_WINDOW, D), index_map=lambda i: (i, 0))],
            core_axis_name=("core", "subcore"),
            dimension_semantics=(pltpu.PARALLEL,),
        )(i_hbm, o_hbm)

    out = gather_kernel(table, flat_idx)
    return out.reshape(B, H, D)


# speedup vs baseline: 1.0977x; 1.0540x over previous
import jax
import jax.numpy as jnp
from jax.experimental import pallas as pl
from jax.experimental.pallas import tpu as pltpu
from jax.experimental.pallas import tpu_sc as plsc

_WINDOW = 512


def kernel(indices, table):
    B, H = indices.shape
    V, D = table.shape
    n = B * H
    flat_idx = indices.reshape(1, n).astype(jnp.int32)
    mesh = plsc.VectorSubcoreMesh(core_axis_name="core", subcore_axis_name="subcore")

    @pl.kernel(
        out_type=jax.ShapeDtypeStruct((n, D), table.dtype),
        mesh=mesh,
        compiler_params=pltpu.CompilerParams(use_tc_tiling_on_sc=False),
    )
    def gather_kernel(x_hbm, i_hbm, o_hbm):
        def body(i_vmem, o_vmem):
            pltpu.sync_copy(x_hbm.at[i_vmem.at[0]], o_vmem)

        pltpu.emit_pipeline(
            body,
            grid=(n // _WINDOW,),
            in_specs=[pl.BlockSpec((1, _WINDOW), index_map=lambda i: (0, i))],
            out_specs=[pl.BlockSpec((_WINDOW, D), index_map=lambda i: (i, 0))],
            core_axis_name=("core", "subcore"),
            dimension_semantics=(pltpu.PARALLEL,),
        )(i_hbm, o_hbm)

    out = gather_kernel(table, flat_idx)
    return out.reshape(B, H, D)


# speedup vs baseline: 1.7127x; 1.5602x over previous
import jax
import jax.numpy as jnp
from jax.experimental import pallas as pl
from jax.experimental.pallas import tpu as pltpu
from jax.experimental.pallas import tpu_sc as plsc
from jax import lax

_NW = 32
_NBUF = 4


def kernel(indices, table):
    B, H = indices.shape
    V, D = table.shape
    idx = indices.astype(jnp.int32)
    per_w = B // _NW

    mesh = plsc.VectorSubcoreMesh(core_axis_name="core", subcore_axis_name="subcore")

    @pl.kernel(
        out_type=jax.ShapeDtypeStruct((B, H, D), table.dtype),
        mesh=mesh,
        compiler_params=pltpu.CompilerParams(use_tc_tiling_on_sc=False),
        scratch_types=(
            [pltpu.VMEM((per_w, H), jnp.int32)]
            + [pltpu.VMEM((H, D), jnp.float32) for _ in range(_NBUF)]
            + [pltpu.SemaphoreType.DMA for _ in range(2 * _NBUF + 1)]
        ),
    )
    def gather_kernel(x_hbm, i_hbm, o_hbm, idx_v, *rest):
        bufs = rest[:_NBUF]
        gsems = rest[_NBUF:2 * _NBUF]
        wsems = rest[2 * _NBUF:3 * _NBUF]
        isem = rest[3 * _NBUF]
        wid = lax.axis_index("subcore") * 2 + lax.axis_index("core")
        base = wid * per_w
        pltpu.async_copy(i_hbm.at[pl.ds(base, per_w)], idx_v, isem).wait()

        def start_gather(row, b):
            pltpu.async_copy(x_hbm.at[idx_v.at[row]], bufs[b], gsems[b])

        for b in range(_NBUF):
            start_gather(b, b)

        @pl.loop(0, per_w, step=_NBUF)
        def _(r):
            for b in range(_NBUF):
                cur = r + b
                pltpu.make_async_copy(x_hbm.at[idx_v.at[0]], bufs[b], gsems[b]).wait()
                pltpu.async_copy(bufs[b], o_hbm.at[base + cur], wsems[b])

                @pl.when(cur + _NBUF < per_w)
                def _():
                    pltpu.make_async_copy(bufs[b], o_hbm.at[0], wsems[b]).wait()
                    start_gather(cur + _NBUF, b)

        for b in range(_NBUF):
            pltpu.make_async_copy(bufs[b], o_hbm.at[0], wsems[b]).wait()

    return gather_kernel(table, idx)


# speedup vs baseline: 1.9414x; 1.1336x over previous
import jax
import jax.numpy as jnp
from jax.experimental import pallas as pl
from jax.experimental.pallas import tpu as pltpu
from jax.experimental.pallas import tpu_sc as plsc
from jax import lax

_NW = 32
_NBUF = 2


def kernel(indices, table):
    B, H = indices.shape
    V, D = table.shape
    idx_t = jnp.transpose(indices).astype(jnp.int32)
    per_w = B // _NW

    mesh = plsc.VectorSubcoreMesh(core_axis_name="core", subcore_axis_name="subcore")

    @pl.kernel(
        out_type=jax.ShapeDtypeStruct((H, B, D), table.dtype),
        mesh=mesh,
        compiler_params=pltpu.CompilerParams(use_tc_tiling_on_sc=False),
        scratch_types=(
            [pltpu.VMEM((H, per_w), jnp.int32)]
            + [pltpu.VMEM((per_w, D), jnp.float32) for _ in range(_NBUF)]
            + [pltpu.SemaphoreType.DMA for _ in range(2 * _NBUF + 1)]
        ),
    )
    def gather_kernel(x_hbm, i_hbm, o_hbm, idx_v, *rest):
        bufs = rest[:_NBUF]
        gsems = rest[_NBUF:2 * _NBUF]
        wsems = rest[2 * _NBUF:3 * _NBUF]
        isem = rest[3 * _NBUF]
        wid = lax.axis_index("subcore") * 2 + lax.axis_index("core")
        base = wid * per_w
        pltpu.async_copy(i_hbm.at[:, pl.ds(base, per_w)], idx_v, isem).wait()

        def start_gather(h, b):
            pltpu.async_copy(x_hbm.at[idx_v.at[h]], bufs[b], gsems[b])

        for b in range(_NBUF):
            start_gather(b, b)

        @pl.loop(0, H, step=_NBUF)
        def _(r):
            for b in range(_NBUF):
                cur = r + b
                pltpu.make_async_copy(x_hbm.at[idx_v.at[0]], bufs[b], gsems[b]).wait()
                pltpu.async_copy(bufs[b], o_hbm.at[cur, pl.ds(base, per_w)], wsems[b])

                @pl.when(cur + _NBUF < H)
                def _():
                    pltpu.make_async_copy(
                        bufs[b], o_hbm.at[0, pl.ds(base, per_w)], wsems[b]).wait()
                    start_gather(cur + _NBUF, b)

        for b in range(_NBUF):
            pltpu.make_async_copy(bufs[b], o_hbm.at[0, pl.ds(base, per_w)], wsems[b]).wait()

    out_t = gather_kernel(table, idx_t)
    return jnp.transpose(out_t, (1, 0, 2))
